# baseline (speedup 1.0000x reference)
import jax
import jax.numpy as jnp
from jax import lax
from jax.experimental import pallas as pl
from jax.experimental.pallas import tpu as pltpu
from jax.experimental.pallas import tpu_sc as plsc

_D = 1024
_B = 8192
_NC = 2
_NS = 16
_NW = _NC * _NS
_BPW = _B // _NW
_SLOT = 16
_NBUF = 7
_RA = 5
_CHUNKS = [8] + [16] * 15 + [8]
_OFFS = [sum(_CHUNKS[:i]) for i in range(len(_CHUNKS))]
_NCHUNK = len(_CHUNKS)


def _sc_gather(idx_hbm, table_hbm, out_hbm, idx_v, rows_v, *sems):
    s_in = sems[:_NBUF]
    s_out = sems[_NBUF:]
    wid = lax.axis_index("s") * _NC + lax.axis_index("c")
    base = wid * _BPW
    pltpu.sync_copy(idx_hbm.at[0, pl.ds(base, _BPW)], idx_v)

    gathers = [None] * _NBUF
    stores = [None] * _NBUF

    def gather(c):
        b = c % _NBUF
        gathers[b] = pltpu.async_copy(
            table_hbm.at[idx_v.at[pl.ds(_OFFS[c], _CHUNKS[c])]],
            rows_v.at[b, pl.ds(0, _CHUNKS[c])], s_in[b])

    for c in range(min(_RA, _NCHUNK)):
        gather(c)
    for c in range(_NCHUNK):
        b = c % _NBUF
        gathers[b].wait()
        stores[b] = pltpu.async_copy(
            rows_v.at[b, pl.ds(0, _CHUNKS[c])],
            out_hbm.at[pl.ds(base + _OFFS[c], _CHUNKS[c])], s_out[b])
        n = c + _RA
        if n < _NCHUNK:
            bn = n % _NBUF
            if stores[bn] is not None:
                stores[bn].wait()
            gather(n)
    for c in range(max(0, _NCHUNK - _NBUF), _NCHUNK):
        stores[c % _NBUF].wait()


@jax.jit
def _gather(idx, table):
    mesh = plsc.VectorSubcoreMesh(core_axis_name="c", subcore_axis_name="s")
    return pl.kernel(
        _sc_gather,
        mesh=mesh,
        out_type=jax.ShapeDtypeStruct((_B, _D), jnp.float32),
        scratch_types=[
            pltpu.VMEM((_BPW,), jnp.int32),
            pltpu.VMEM((_NBUF, _SLOT, _D), jnp.float32),
        ] + [pltpu.SemaphoreType.DMA] * (2 * _NBUF),
    )(idx, table)


def kernel(positions, P):
    out = _gather(positions.astype(jnp.int32), P[0])
    return out[None]


# speedup vs baseline: 1.0159x; 1.0159x over previous
import jax
import jax.numpy as jnp
from jax import lax
from jax.experimental import pallas as pl
from jax.experimental.pallas import tpu as pltpu
from jax.experimental.pallas import tpu_sc as plsc

_D = 1024
_B = 8192
_NC = 2
_NS = 16
_NW = _NC * _NS
_BPW = _B // _NW
_SLOT = 16
_NBUF = 7
_RA = 5
_CHUNKS = [16] * 16
_OFFS = [sum(_CHUNKS[:i]) for i in range(len(_CHUNKS))]
_NCHUNK = len(_CHUNKS)


def _sc_gather(idx_hbm, table_hbm, out_hbm, idx_v, rows_v, *sems):
    s_in = sems[:_NBUF]
    s_out = sems[_NBUF:]
    wid = lax.axis_index("s") * _NC + lax.axis_index("c")
    base = wid * _BPW
    pltpu.sync_copy(idx_hbm.at[0, pl.ds(base, _BPW)], idx_v)

    gathers = [None] * _NBUF
    stores = [None] * _NBUF

    def gather(c):
        b = c % _NBUF
        gathers[b] = pltpu.async_copy(
            table_hbm.at[idx_v.at[pl.ds(_OFFS[c], _CHUNKS[c])]],
            rows_v.at[b, pl.ds(0, _CHUNKS[c])], s_in[b])

    for c in range(min(_RA, _NCHUNK)):
        gather(c)
    for c in range(_NCHUNK):
        b = c % _NBUF
        gathers[b].wait()
        stores[b] = pltpu.async_copy(
            rows_v.at[b, pl.ds(0, _CHUNKS[c])],
            out_hbm.at[pl.ds(base + _OFFS[c], _CHUNKS[c])], s_out[b])
        n = c + _RA
        if n < _NCHUNK:
            bn = n % _NBUF
            if stores[bn] is not None:
                stores[bn].wait()
            gather(n)
    for c in range(max(0, _NCHUNK - _NBUF), _NCHUNK):
        stores[c % _NBUF].wait()


@jax.jit
def _gather(idx, table):
    mesh = plsc.VectorSubcoreMesh(core_axis_name="c", subcore_axis_name="s")
    return pl.kernel(
        _sc_gather,
        mesh=mesh,
        out_type=jax.ShapeDtypeStruct((_B, _D), jnp.float32),
        scratch_types=[
            pltpu.VMEM((_BPW,), jnp.int32),
            pltpu.VMEM((_NBUF, _SLOT, _D), jnp.float32),
        ] + [pltpu.SemaphoreType.DMA] * (2 * _NBUF),
    )(idx, table)


def kernel(positions, P):
    out = _gather(positions.astype(jnp.int32), P[0])
    return out[None]
